# baseline (speedup 1.0000x reference)
import functools

import numpy as np

import jax
import jax.numpy as jnp
from jax import lax
from jax.experimental import pallas as pl
from jax.experimental.pallas import tpu as pltpu
from jax.experimental.pallas import tpu_sc as plsc

V = 8192
NX = 2048
NY = 2048
L = 16
NC = 2
NS = 16
NW = NC * NS
B = 4
NBUF = 3
SHARE0 = 14
SHARE1 = 18
NBMAX = SHARE1
ROWS0 = SHARE0 * B
ROWS1 = SHARE1 * B
SPLIT = NS * ROWS0

assert NS * (ROWS0 + ROWS1) == NX

_pos = np.zeros((NW, NBMAX * B), np.int32)
for _w in range(NW):
    _sid, _cid = _w // NC, _w % NC
    if _cid == 0:
        _b0, _n = _sid * ROWS0, ROWS0
    else:
        _b0, _n = SPLIT + _sid * ROWS1, ROWS1
    _pos[_w, :_n] = np.arange(_b0, _b0 + _n)
_POS = _pos.reshape(NW, NBMAX, B)


def _sc_body(dist_hbm, x_hbm, y_hbm, nis_hbm, out_hbm,
             xv, yv, nisv, rows, outb, isems, osems):
    cid = lax.axis_index("c")
    sid = lax.axis_index("s")
    wid = sid * NC + cid
    is1 = cid == 1
    base = jnp.where(is1, SPLIT + sid * ROWS1, sid * ROWS0)

    pltpu.sync_copy(x_hbm.at[wid], xv)
    pltpu.sync_copy(y_hbm, yv)
    pltpu.sync_copy(nis_hbm, nisv)
    nis = nisv[...]

    rsels = [jnp.full((L,), r, jnp.int32) for r in range(B)]

    def fire_in(k):
        pltpu.async_copy(dist_hbm.at[xv.at[k]], rows.at[k % NBUF],
                         isems[k % NBUF])

    def wait_in(k):
        pltpu.make_async_copy(dist_hbm.at[xv.at[k]], rows.at[k % NBUF],
                              isems[k % NBUF]).wait()

    def fire_out(k):
        pltpu.async_copy(outb.at[k % NBUF],
                         out_hbm.at[pl.ds(base + k * B, B)], osems[k % NBUF])

    def wait_out(slot):
        pltpu.make_async_copy(outb.at[slot], out_hbm.at[pl.ds(0, B)],
                              osems[slot]).wait()

    def guarded(k, fn):
        if k < SHARE0:
            fn()
        else:
            pl.when(is1)(fn)

    for k in range(NBUF - 1):
        guarded(k, functools.partial(fire_in, k))

    for k in range(NBMAX):
        buf = k % NBUF
        if k + NBUF - 1 < NBMAX:
            guarded(k + NBUF - 1, functools.partial(fire_in, k + NBUF - 1))

        def step(k=k, buf=buf):
            wait_in(k)
            if k >= NBUF:
                wait_out(buf)

            @plsc.parallel_loop(0, NY, step=L, unroll=8)
            def col_body(jj):
                idx = yv[pl.ds(jj, L)]
                for r in range(B):
                    g = plsc.load_gather(rows.at[buf], [rsels[r], idx])
                    outb[buf, r, pl.ds(jj, L)] = jnp.exp(g * nis)
            fire_out(k)

        guarded(k, step)

    for slot in range(NBUF):
        wait_out(slot)


_sc_call = functools.partial(
    pl.kernel,
    out_type=jax.ShapeDtypeStruct((NX, NY), jnp.float32),
    mesh=plsc.VectorSubcoreMesh(
        core_axis_name="c", subcore_axis_name="s",
        num_cores=NC, num_subcores=NS),
    scratch_types=[
        pltpu.VMEM((NBMAX, B), jnp.int32),
        pltpu.VMEM((NY,), jnp.int32),
        pltpu.VMEM((L,), jnp.float32),
        pltpu.VMEM((NBUF, B, V), jnp.float32),
        pltpu.VMEM((NBUF, B, NY), jnp.float32),
        [pltpu.SemaphoreType.DMA] * NBUF,
        [pltpu.SemaphoreType.DMA] * NBUF,
    ],
    compiler_params=pltpu.CompilerParams(
        use_tc_tiling_on_sc=True, needs_layout_passes=False),
)(_sc_body)


def kernel(x, y, distance, scale):
    s = jnp.clip(jax.nn.softplus(scale), 1e-10, 10000.0)
    nis = jnp.broadcast_to((-1.0 / s).astype(jnp.float32), (L,))
    xp = x.astype(jnp.int32)[_POS]
    return _sc_call(distance, xp, y.astype(jnp.int32), nis)
